# Initial kernel scaffold; baseline (speedup 1.0000x reference)
import jax
import jax.numpy as jnp
from jax.experimental import pallas as pl


def kernel(x, edge_index, W1, a_src1, a_dst1, b1, gamma, beta, W2, a_src2, a_dst2, b2):
    raise NotImplementedError("write your pallas kernel here")


# speedup vs baseline: 26.9462x; 26.9462x over previous
import functools

import jax
import jax.numpy as jnp
import numpy as np
from jax import lax
from jax.experimental import pallas as pl
from jax.experimental.pallas import tpu as pltpu
from jax.experimental.pallas import tpu_sc as plsc

N = 10000
E = 320000
D_IN = 128
HID = 64
OUT = 128
H = 2

R = 10240
BLK = 1024
GRID = R // BLK
CHUNK = 128
NSUB = 16
E1 = E + N
EPAD = ((E1 + NSUB * CHUNK - 1) // (NSUB * CHUNK)) * (NSUB * CHUNK)
PER_TILE = EPAD // NSUB
NCHUNK = PER_TILE // CHUNK
TW = 80
_DEV_STAGE = 4


def _ones_pad_cols(nrows, width):
    col = lax.broadcasted_iota(jnp.int32, (nrows, width), 1)
    return jnp.where(col == 0, 1.0, 0.0).astype(jnp.float32)


def _mk_tab(h64):
    return jnp.concatenate([h64, _ones_pad_cols(BLK, TW - HID)], axis=1)


def _stage_a_body(x_ref, w_ref, a_ref, tab_ref, al_ref):
    h = jnp.dot(x_ref[...], w_ref[...], preferred_element_type=jnp.float32)
    al_ref[...] = jnp.dot(h, a_ref[...], preferred_element_type=jnp.float32)
    tab_ref[...] = jnp.stack([_mk_tab(h[:, :HID]), _mk_tab(h[:, HID:])])


def _stage_a(xp, W1, A1):
    return pl.pallas_call(
        _stage_a_body,
        grid=(GRID,),
        in_specs=[
            pl.BlockSpec((BLK, D_IN), lambda i: (i, 0)),
            pl.BlockSpec((D_IN, H * HID), lambda i: (0, 0)),
            pl.BlockSpec((H * HID, 8), lambda i: (0, 0)),
        ],
        out_specs=[
            pl.BlockSpec((H, BLK, TW), lambda i: (0, i, 0)),
            pl.BlockSpec((BLK, 8), lambda i: (i, 0)),
        ],
        out_shape=[
            jax.ShapeDtypeStruct((H, R, TW), jnp.float32),
            jax.ShapeDtypeStruct((R, 8), jnp.float32),
        ],
    )(xp, W1, A1)


def _stage_c_body(acc_ref, b1_ref, g_ref, be_ref, w2_ref, a2_ref,
                  taba_ref, tabb_ref, al_ref):
    a0 = acc_ref[0]
    a1 = acc_ref[1]
    h0 = a0[:, :HID] / (a0[:, HID:HID + 1] + 1e-16)
    h1 = a1[:, :HID] / (a1[:, HID:HID + 1] + 1e-16)
    h = jnp.concatenate([h0, h1], axis=1) + b1_ref[...]
    h = h * (g_ref[...] * np.float32(1.0 / np.sqrt(1.0 + 1e-5))) + be_ref[...]
    h = jnp.where(h > 0, h, jnp.exp(h) - 1.0)
    h2 = jnp.dot(h, w2_ref[...], preferred_element_type=jnp.float32)
    al_ref[...] = jnp.dot(h2, a2_ref[...], preferred_element_type=jnp.float32)
    taba_ref[...] = jnp.stack([_mk_tab(h2[:, :HID]),
                               _mk_tab(h2[:, OUT:OUT + HID])])
    tabb_ref[...] = jnp.stack([_mk_tab(h2[:, HID:OUT]),
                               _mk_tab(h2[:, OUT + HID:])])


def _stage_c(acc1, b1, gamma, beta, W2, A2):
    return pl.pallas_call(
        _stage_c_body,
        grid=(GRID,),
        in_specs=[
            pl.BlockSpec((H, BLK, TW), lambda i: (0, i, 0)),
            pl.BlockSpec((1, H * HID), lambda i: (0, 0)),
            pl.BlockSpec((1, H * HID), lambda i: (0, 0)),
            pl.BlockSpec((1, H * HID), lambda i: (0, 0)),
            pl.BlockSpec((H * HID, H * OUT), lambda i: (0, 0)),
            pl.BlockSpec((H * OUT, 8), lambda i: (0, 0)),
        ],
        out_specs=[
            pl.BlockSpec((H, BLK, TW), lambda i: (0, i, 0)),
            pl.BlockSpec((H, BLK, TW), lambda i: (0, i, 0)),
            pl.BlockSpec((BLK, 8), lambda i: (i, 0)),
        ],
        out_shape=[
            jax.ShapeDtypeStruct((H, R, TW), jnp.float32),
            jax.ShapeDtypeStruct((H, R, TW), jnp.float32),
            jax.ShapeDtypeStruct((R, 8), jnp.float32),
        ],
    )(acc1, b1, gamma, beta, W2, A2)


def _stage_e_body(acca_ref, accb_ref, b2_ref, out_ref):
    oa = [acca_ref[c][:, :HID] / (acca_ref[c][:, HID:HID + 1] + 1e-16)
          for c in range(H)]
    ob = [accb_ref[c][:, :HID] / (accb_ref[c][:, HID:HID + 1] + 1e-16)
          for c in range(H)]
    lo = 0.5 * (oa[0] + oa[1])
    hi = 0.5 * (ob[0] + ob[1])
    out_ref[...] = jnp.concatenate([lo, hi], axis=1) + b2_ref[...]


def _stage_e(acc2a, acc2b, b2):
    return pl.pallas_call(
        _stage_e_body,
        grid=(GRID,),
        in_specs=[
            pl.BlockSpec((H, BLK, TW), lambda i: (0, i, 0)),
            pl.BlockSpec((H, BLK, TW), lambda i: (0, i, 0)),
            pl.BlockSpec((1, OUT), lambda i: (0, 0)),
        ],
        out_specs=pl.BlockSpec((BLK, OUT), lambda i: (i, 0)),
        out_shape=jax.ShapeDtypeStruct((R, OUT), jnp.float32),
    )(acc2a, acc2b, b2)


def _edge_body(srcs, dsts, tab, asrc, adst, out,
               idxs_v, idxd_v, w_v, rows_v, as_v, ad_v, acc, sem):
    c = lax.axis_index("c")
    s = lax.axis_index("s")
    c_off = c * R
    rows_per_tile = R // NSUB
    nv = TW // 16

    if _DEV_STAGE <= -1:
        def zb(i, carry):
            for r in range(nv):
                rows_v[i, pl.ds(r * 16, 16)] = jnp.zeros((16,), jnp.float32)
            return carry
        lax.fori_loop(0, CHUNK, zb, 0)
        if _DEV_STAGE == -3:
            pltpu.sync_copy(rows_v, out.at[pl.ds((c * NSUB + s) * CHUNK, CHUNK)])
            return
        if _DEV_STAGE == -2:
            pltpu.sync_copy(rows_v, acc.at[pl.ds(s * CHUNK, CHUNK)])
            pltpu.sync_copy(acc.at[pl.ds(s * CHUNK, CHUNK)],
                            out.at[pl.ds((c * NSUB + s) * CHUNK, CHUNK)])
            return
        pltpu.sync_copy(rows_v, acc.at[pl.ds(s * CHUNK, CHUNK)])
        plsc.subcore_barrier()
        pltpu.sync_copy(acc.at[pl.ds(s * CHUNK, CHUNK)],
                        out.at[pl.ds((c * NSUB + s) * CHUNK, CHUNK)])
        return

    def zbody(i, carry):
        for r in range(nv):
            rows_v[i, pl.ds(r * 16, 16)] = jnp.zeros((16,), jnp.float32)
        return carry
    lax.fori_loop(0, CHUNK, zbody, 0)
    for j in range(rows_per_tile // CHUNK):
        pltpu.sync_copy(
            rows_v, acc.at[pl.ds(s * rows_per_tile + j * CHUNK, CHUNK)])

    pltpu.sync_copy(asrc, as_v)
    pltpu.sync_copy(adst, ad_v)
    plsc.subcore_barrier()

    def chunk_body(k, carry):
        base = s * PER_TILE + k * CHUNK
        pltpu.sync_copy(srcs.at[pl.ds(base, CHUNK)], idxs_v)
        pltpu.sync_copy(dsts.at[pl.ds(base, CHUNK)], idxd_v)
        if _DEV_STAGE >= 2:
            for i in range(CHUNK // 16):
                s16 = idxs_v[pl.ds(i * 16, 16)] + c_off
                idxs_v[pl.ds(i * 16, 16)] = s16
                d16 = idxd_v[pl.ds(i * 16, 16)] + c_off
                a = plsc.load_gather(as_v, [s16]) + plsc.load_gather(ad_v, [d16])
                a = jnp.maximum(a, 0.2 * a)
                w_v[pl.ds(i * 16, 16)] = jnp.exp(a)
        if _DEV_STAGE >= 3:
            pltpu.async_copy(tab.at[idxs_v], rows_v, sem).wait()

            def mul_body(e, carry2):
                wb = plsc.load_gather(w_v, [jnp.broadcast_to(e, (16,))])
                for r in range(nv):
                    rows_v[e, pl.ds(r * 16, 16)] = (
                        rows_v[e, pl.ds(r * 16, 16)] * wb)
                return carry2
            lax.fori_loop(0, CHUNK, mul_body, 0)
        if _DEV_STAGE >= 4:
            pltpu.sync_copy(rows_v, acc.at[idxd_v], add=True)
        return carry
    if _DEV_STAGE >= 1:
        lax.fori_loop(0, NCHUNK, chunk_body, 0)

    plsc.subcore_barrier()
    pltpu.sync_copy(
        acc.at[pl.ds(s * rows_per_tile, rows_per_tile)],
        out.at[pl.ds(c_off + s * rows_per_tile, rows_per_tile)])


def _edge_pass():
    mesh = plsc.VectorSubcoreMesh(
        core_axis_name="c", subcore_axis_name="s", num_cores=H,
        num_subcores=NSUB)
    return pl.kernel(
        _edge_body,
        out_type=jax.ShapeDtypeStruct((H * R, TW), jnp.float32),
        mesh=mesh,
        scratch_types=[
            pltpu.VMEM((CHUNK,), jnp.int32),
            pltpu.VMEM((CHUNK,), jnp.int32),
            pltpu.VMEM((CHUNK,), jnp.float32),
            pltpu.VMEM((CHUNK, TW), jnp.float32),
            pltpu.VMEM((H * R,), jnp.float32),
            pltpu.VMEM((H * R,), jnp.float32),
            pltpu.VMEM_SHARED((R, TW), jnp.float32),
            pltpu.SemaphoreType.DMA,
        ],
        compiler_params=pltpu.CompilerParams(
            needs_layout_passes=False, use_tc_tiling_on_sc=False),
    )


def kernel(x, edge_index, W1, a_src1, a_dst1, b1, gamma, beta, W2, a_src2, a_dst2, b2):
    f32 = jnp.float32
    i32 = jnp.int32
    xp = jnp.zeros((R, D_IN), f32).at[:N].set(x)
    loop = jnp.arange(N, dtype=i32)
    padi = jnp.full((EPAD - E1,), N, dtype=i32)
    srcs = jnp.concatenate([edge_index[0].astype(i32), loop, padi])
    dsts = jnp.concatenate([edge_index[1].astype(i32), loop, padi])

    A1 = jnp.zeros((H * HID, 8), f32)
    A1 = A1.at[:HID, 0].set(a_src1[0]).at[HID:, 1].set(a_src1[1])
    A1 = A1.at[:HID, 2].set(a_dst1[0]).at[HID:, 3].set(a_dst1[1])
    A2 = jnp.zeros((H * OUT, 8), f32)
    A2 = A2.at[:OUT, 0].set(a_src2[0]).at[OUT:, 1].set(a_src2[1])
    A2 = A2.at[:OUT, 2].set(a_dst2[0]).at[OUT:, 3].set(a_dst2[1])

    edge = _edge_pass()

    tab1, al1 = _stage_a(xp, W1, A1)
    asrc1 = jnp.concatenate([al1[:, 0], al1[:, 1]])
    adst1 = jnp.concatenate([al1[:, 2], al1[:, 3]])
    acc1 = edge(srcs, dsts, tab1.reshape(H * R, TW), asrc1, adst1)

    tab2a, tab2b, al2 = _stage_c(acc1.reshape(H, R, TW), b1.reshape(1, -1),
                                 gamma.reshape(1, -1), beta.reshape(1, -1),
                                 W2, A2)
    asrc2 = jnp.concatenate([al2[:, 0], al2[:, 1]])
    adst2 = jnp.concatenate([al2[:, 2], al2[:, 3]])
    acc2a = edge(srcs, dsts, tab2a.reshape(H * R, TW), asrc2, adst2)
    acc2b = edge(srcs, dsts, tab2b.reshape(H * R, TW), asrc2, adst2)

    out = _stage_e(acc2a.reshape(H, R, TW), acc2b.reshape(H, R, TW),
                   b2.reshape(1, -1))
    return out[:N]


# speedup vs baseline: 26.9642x; 1.0007x over previous
import functools

import jax
import jax.numpy as jnp
import numpy as np
from jax import lax
from jax.experimental import pallas as pl
from jax.experimental.pallas import tpu as pltpu
from jax.experimental.pallas import tpu_sc as plsc

N = 10000
E = 320000
D_IN = 128
HID = 64
OUT = 128
H = 2

R = 10240
BLK = 1024
GRID = R // BLK
CHUNK = 128
NSUB = 16
E1 = E + N
EPAD = ((E1 + NSUB * CHUNK - 1) // (NSUB * CHUNK)) * (NSUB * CHUNK)
PER_TILE = EPAD // NSUB
NCHUNK = PER_TILE // CHUNK
TW = 80


def _ones_pad_cols(nrows, width):
    col = lax.broadcasted_iota(jnp.int32, (nrows, width), 1)
    return jnp.where(col == 0, 1.0, 0.0).astype(jnp.float32)


def _mk_tab(h64):
    return jnp.concatenate([h64, _ones_pad_cols(BLK, TW - HID)], axis=1)


def _stage_a_body(x_ref, w_ref, a_ref, tab_ref, al_ref):
    h = jnp.dot(x_ref[...], w_ref[...], preferred_element_type=jnp.float32)
    al_ref[...] = jnp.dot(h, a_ref[...], preferred_element_type=jnp.float32)
    tab_ref[...] = jnp.stack([_mk_tab(h[:, :HID]), _mk_tab(h[:, HID:])])


def _stage_a(xp, W1, A1):
    return pl.pallas_call(
        _stage_a_body,
        grid=(GRID,),
        in_specs=[
            pl.BlockSpec((BLK, D_IN), lambda i: (i, 0)),
            pl.BlockSpec((D_IN, H * HID), lambda i: (0, 0)),
            pl.BlockSpec((H * HID, 8), lambda i: (0, 0)),
        ],
        out_specs=[
            pl.BlockSpec((H, BLK, TW), lambda i: (0, i, 0)),
            pl.BlockSpec((BLK, 8), lambda i: (i, 0)),
        ],
        out_shape=[
            jax.ShapeDtypeStruct((H, R, TW), jnp.float32),
            jax.ShapeDtypeStruct((R, 8), jnp.float32),
        ],
    )(xp, W1, A1)


def _stage_c_body(acc_ref, b1_ref, g_ref, be_ref, w2_ref, a2_ref,
                  taba_ref, tabb_ref, al_ref):
    a0 = acc_ref[0]
    a1 = acc_ref[1]
    h0 = a0[:, :HID] / (a0[:, HID:HID + 1] + 1e-16)
    h1 = a1[:, :HID] / (a1[:, HID:HID + 1] + 1e-16)
    h = jnp.concatenate([h0, h1], axis=1) + b1_ref[...]
    h = h * (g_ref[...] * np.float32(1.0 / np.sqrt(1.0 + 1e-5))) + be_ref[...]
    h = jnp.where(h > 0, h, jnp.exp(h) - 1.0)
    h2 = jnp.dot(h, w2_ref[...], preferred_element_type=jnp.float32)
    al_ref[...] = jnp.dot(h2, a2_ref[...], preferred_element_type=jnp.float32)
    taba_ref[...] = jnp.stack([_mk_tab(h2[:, :HID]),
                               _mk_tab(h2[:, OUT:OUT + HID])])
    tabb_ref[...] = jnp.stack([_mk_tab(h2[:, HID:OUT]),
                               _mk_tab(h2[:, OUT + HID:])])


def _stage_c(acc1, b1, gamma, beta, W2, A2):
    return pl.pallas_call(
        _stage_c_body,
        grid=(GRID,),
        in_specs=[
            pl.BlockSpec((H, BLK, TW), lambda i: (0, i, 0)),
            pl.BlockSpec((1, H * HID), lambda i: (0, 0)),
            pl.BlockSpec((1, H * HID), lambda i: (0, 0)),
            pl.BlockSpec((1, H * HID), lambda i: (0, 0)),
            pl.BlockSpec((H * HID, H * OUT), lambda i: (0, 0)),
            pl.BlockSpec((H * OUT, 8), lambda i: (0, 0)),
        ],
        out_specs=[
            pl.BlockSpec((H, BLK, TW), lambda i: (0, i, 0)),
            pl.BlockSpec((H, BLK, TW), lambda i: (0, i, 0)),
            pl.BlockSpec((BLK, 8), lambda i: (i, 0)),
        ],
        out_shape=[
            jax.ShapeDtypeStruct((H, R, TW), jnp.float32),
            jax.ShapeDtypeStruct((H, R, TW), jnp.float32),
            jax.ShapeDtypeStruct((R, 8), jnp.float32),
        ],
    )(acc1, b1, gamma, beta, W2, A2)


def _stage_e_body(acca_ref, accb_ref, b2_ref, out_ref):
    oa = [acca_ref[c][:, :HID] / (acca_ref[c][:, HID:HID + 1] + 1e-16)
          for c in range(H)]
    ob = [accb_ref[c][:, :HID] / (accb_ref[c][:, HID:HID + 1] + 1e-16)
          for c in range(H)]
    lo = 0.5 * (oa[0] + oa[1])
    hi = 0.5 * (ob[0] + ob[1])
    out_ref[...] = jnp.concatenate([lo, hi], axis=1) + b2_ref[...]


def _stage_e(acc2a, acc2b, b2):
    return pl.pallas_call(
        _stage_e_body,
        grid=(GRID,),
        in_specs=[
            pl.BlockSpec((H, BLK, TW), lambda i: (0, i, 0)),
            pl.BlockSpec((H, BLK, TW), lambda i: (0, i, 0)),
            pl.BlockSpec((1, OUT), lambda i: (0, 0)),
        ],
        out_specs=pl.BlockSpec((BLK, OUT), lambda i: (i, 0)),
        out_shape=jax.ShapeDtypeStruct((R, OUT), jnp.float32),
    )(acc2a, acc2b, b2)


def _edge_body(srcs, dsts, tab, asrc, adst, out,
               idxs_v, idxd_v, w_v, rows_v, as_v, ad_v, acc, sem):
    c = lax.axis_index("c")
    s = lax.axis_index("s")
    c_off = c * R
    rows_per_tile = R // NSUB
    nv = TW // 16

    def zbody(i, carry):
        for r in range(nv):
            rows_v[i, pl.ds(r * 16, 16)] = jnp.zeros((16,), jnp.float32)
        return carry
    lax.fori_loop(0, CHUNK, zbody, 0)
    for j in range(rows_per_tile // CHUNK):
        pltpu.sync_copy(
            rows_v, acc.at[pl.ds(s * rows_per_tile + j * CHUNK, CHUNK)])

    pltpu.sync_copy(asrc, as_v)
    pltpu.sync_copy(adst, ad_v)
    plsc.subcore_barrier()

    def chunk_body(k, carry):
        base = s * PER_TILE + k * CHUNK
        pltpu.sync_copy(srcs.at[pl.ds(base, CHUNK)], idxs_v)
        pltpu.sync_copy(dsts.at[pl.ds(base, CHUNK)], idxd_v)
        for i in range(CHUNK // 16):
            s16 = idxs_v[pl.ds(i * 16, 16)] + c_off
            idxs_v[pl.ds(i * 16, 16)] = s16
            d16 = idxd_v[pl.ds(i * 16, 16)] + c_off
            a = plsc.load_gather(as_v, [s16]) + plsc.load_gather(ad_v, [d16])
            a = jnp.maximum(a, 0.2 * a)
            w_v[pl.ds(i * 16, 16)] = jnp.exp(a)
        pltpu.async_copy(tab.at[idxs_v], rows_v, sem).wait()

        def mul_body(e, carry2):
            wb = plsc.load_gather(w_v, [jnp.broadcast_to(e, (16,))])
            for r in range(nv):
                rows_v[e, pl.ds(r * 16, 16)] = (
                    rows_v[e, pl.ds(r * 16, 16)] * wb)
            return carry2
        lax.fori_loop(0, CHUNK, mul_body, 0)
        pltpu.sync_copy(rows_v, acc.at[idxd_v], add=True)
        return carry
    lax.fori_loop(0, NCHUNK, chunk_body, 0)

    plsc.subcore_barrier()
    pltpu.sync_copy(
        acc.at[pl.ds(s * rows_per_tile, rows_per_tile)],
        out.at[pl.ds(c_off + s * rows_per_tile, rows_per_tile)])


def _edge_pass():
    mesh = plsc.VectorSubcoreMesh(
        core_axis_name="c", subcore_axis_name="s", num_cores=H,
        num_subcores=NSUB)
    return pl.kernel(
        _edge_body,
        out_type=jax.ShapeDtypeStruct((H * R, TW), jnp.float32),
        mesh=mesh,
        scratch_types=[
            pltpu.VMEM((CHUNK,), jnp.int32),
            pltpu.VMEM((CHUNK,), jnp.int32),
            pltpu.VMEM((CHUNK,), jnp.float32),
            pltpu.VMEM((CHUNK, TW), jnp.float32),
            pltpu.VMEM((H * R,), jnp.float32),
            pltpu.VMEM((H * R,), jnp.float32),
            pltpu.VMEM_SHARED((R, TW), jnp.float32),
            pltpu.SemaphoreType.DMA,
        ],
        compiler_params=pltpu.CompilerParams(
            needs_layout_passes=False, use_tc_tiling_on_sc=False),
    )


def kernel(x, edge_index, W1, a_src1, a_dst1, b1, gamma, beta, W2, a_src2, a_dst2, b2):
    f32 = jnp.float32
    i32 = jnp.int32
    xp = jnp.zeros((R, D_IN), f32).at[:N].set(x)
    loop = jnp.arange(N, dtype=i32)
    padi = jnp.full((EPAD - E1,), N, dtype=i32)
    srcs = jnp.concatenate([edge_index[0].astype(i32), loop, padi])
    dsts = jnp.concatenate([edge_index[1].astype(i32), loop, padi])

    A1 = jnp.zeros((H * HID, 8), f32)
    A1 = A1.at[:HID, 0].set(a_src1[0]).at[HID:, 1].set(a_src1[1])
    A1 = A1.at[:HID, 2].set(a_dst1[0]).at[HID:, 3].set(a_dst1[1])
    A2 = jnp.zeros((H * OUT, 8), f32)
    A2 = A2.at[:OUT, 0].set(a_src2[0]).at[OUT:, 1].set(a_src2[1])
    A2 = A2.at[:OUT, 2].set(a_dst2[0]).at[OUT:, 3].set(a_dst2[1])

    edge = _edge_pass()

    tab1, al1 = _stage_a(xp, W1, A1)
    asrc1 = jnp.concatenate([al1[:, 0], al1[:, 1]])
    adst1 = jnp.concatenate([al1[:, 2], al1[:, 3]])
    acc1 = edge(srcs, dsts, tab1.reshape(H * R, TW), asrc1, adst1)

    tab2a, tab2b, al2 = _stage_c(acc1.reshape(H, R, TW), b1.reshape(1, -1),
                                 gamma.reshape(1, -1), beta.reshape(1, -1),
                                 W2, A2)
    asrc2 = jnp.concatenate([al2[:, 0], al2[:, 1]])
    adst2 = jnp.concatenate([al2[:, 2], al2[:, 3]])
    acc2a = edge(srcs, dsts, tab2a.reshape(H * R, TW), asrc2, adst2)
    acc2b = edge(srcs, dsts, tab2b.reshape(H * R, TW), asrc2, adst2)

    out = _stage_e(acc2a.reshape(H, R, TW), acc2b.reshape(H, R, TW),
                   b2.reshape(1, -1))
    return out[:N]


# speedup vs baseline: 38.3537x; 1.4224x over previous
import functools

import jax
import jax.numpy as jnp
import numpy as np
from jax import lax
from jax.experimental import pallas as pl
from jax.experimental.pallas import tpu as pltpu
from jax.experimental.pallas import tpu_sc as plsc

N = 10000
E = 320000
D_IN = 128
HID = 64
OUT = 128
H = 2

R = 10240
BLK = 1024
GRID = R // BLK
CHUNK = 128
NSUB = 16
E1 = E + N
EPAD = ((E1 + NSUB * CHUNK - 1) // (NSUB * CHUNK)) * (NSUB * CHUNK)
PER_TILE = EPAD // NSUB
NCHUNK = PER_TILE // CHUNK
TW = 80


def _ones_pad_cols(nrows, width):
    col = lax.broadcasted_iota(jnp.int32, (nrows, width), 1)
    return jnp.where(col == 0, 1.0, 0.0).astype(jnp.float32)


def _mk_tab(h64):
    return jnp.concatenate([h64, _ones_pad_cols(BLK, TW - HID)], axis=1)


def _stage_a_body(x_ref, w_ref, a_ref, tab_ref, al_ref):
    h = jnp.dot(x_ref[...], w_ref[...], preferred_element_type=jnp.float32)
    al_ref[...] = jnp.dot(h, a_ref[...], preferred_element_type=jnp.float32)
    tab_ref[...] = jnp.stack([_mk_tab(h[:, :HID]), _mk_tab(h[:, HID:])])


def _stage_a(xp, W1, A1):
    return pl.pallas_call(
        _stage_a_body,
        grid=(GRID,),
        in_specs=[
            pl.BlockSpec((BLK, D_IN), lambda i: (i, 0)),
            pl.BlockSpec((D_IN, H * HID), lambda i: (0, 0)),
            pl.BlockSpec((H * HID, 8), lambda i: (0, 0)),
        ],
        out_specs=[
            pl.BlockSpec((H, BLK, TW), lambda i: (0, i, 0)),
            pl.BlockSpec((BLK, 8), lambda i: (i, 0)),
        ],
        out_shape=[
            jax.ShapeDtypeStruct((H, R, TW), jnp.float32),
            jax.ShapeDtypeStruct((R, 8), jnp.float32),
        ],
    )(xp, W1, A1)


def _stage_c_body(acc_ref, b1_ref, g_ref, be_ref, w2_ref, a2_ref,
                  taba_ref, tabb_ref, al_ref):
    a0 = acc_ref[0]
    a1 = acc_ref[1]
    h0 = a0[:, :HID] / (a0[:, HID:HID + 1] + 1e-16)
    h1 = a1[:, :HID] / (a1[:, HID:HID + 1] + 1e-16)
    h = jnp.concatenate([h0, h1], axis=1) + b1_ref[...]
    h = h * (g_ref[...] * np.float32(1.0 / np.sqrt(1.0 + 1e-5))) + be_ref[...]
    h = jnp.where(h > 0, h, jnp.exp(h) - 1.0)
    h2 = jnp.dot(h, w2_ref[...], preferred_element_type=jnp.float32)
    al_ref[...] = jnp.dot(h2, a2_ref[...], preferred_element_type=jnp.float32)
    taba_ref[...] = jnp.stack([_mk_tab(h2[:, :HID]),
                               _mk_tab(h2[:, OUT:OUT + HID])])
    tabb_ref[...] = jnp.stack([_mk_tab(h2[:, HID:OUT]),
                               _mk_tab(h2[:, OUT + HID:])])


def _stage_c(acc1, b1, gamma, beta, W2, A2):
    return pl.pallas_call(
        _stage_c_body,
        grid=(GRID,),
        in_specs=[
            pl.BlockSpec((H, BLK, TW), lambda i: (0, i, 0)),
            pl.BlockSpec((1, H * HID), lambda i: (0, 0)),
            pl.BlockSpec((1, H * HID), lambda i: (0, 0)),
            pl.BlockSpec((1, H * HID), lambda i: (0, 0)),
            pl.BlockSpec((H * HID, H * OUT), lambda i: (0, 0)),
            pl.BlockSpec((H * OUT, 8), lambda i: (0, 0)),
        ],
        out_specs=[
            pl.BlockSpec((H, BLK, TW), lambda i: (0, i, 0)),
            pl.BlockSpec((H, BLK, TW), lambda i: (0, i, 0)),
            pl.BlockSpec((BLK, 8), lambda i: (i, 0)),
        ],
        out_shape=[
            jax.ShapeDtypeStruct((H, R, TW), jnp.float32),
            jax.ShapeDtypeStruct((H, R, TW), jnp.float32),
            jax.ShapeDtypeStruct((R, 8), jnp.float32),
        ],
    )(acc1, b1, gamma, beta, W2, A2)


def _stage_e_body(acca_ref, accb_ref, b2_ref, out_ref):
    oa = [acca_ref[c][:, :HID] / (acca_ref[c][:, HID:HID + 1] + 1e-16)
          for c in range(H)]
    ob = [accb_ref[c][:, :HID] / (accb_ref[c][:, HID:HID + 1] + 1e-16)
          for c in range(H)]
    lo = 0.5 * (oa[0] + oa[1])
    hi = 0.5 * (ob[0] + ob[1])
    out_ref[...] = jnp.concatenate([lo, hi], axis=1) + b2_ref[...]


def _stage_e(acc2a, acc2b, b2):
    return pl.pallas_call(
        _stage_e_body,
        grid=(GRID,),
        in_specs=[
            pl.BlockSpec((H, BLK, TW), lambda i: (0, i, 0)),
            pl.BlockSpec((H, BLK, TW), lambda i: (0, i, 0)),
            pl.BlockSpec((1, OUT), lambda i: (0, 0)),
        ],
        out_specs=pl.BlockSpec((BLK, OUT), lambda i: (i, 0)),
        out_shape=jax.ShapeDtypeStruct((R, OUT), jnp.float32),
    )(acc2a, acc2b, b2)


def _edge_body(srcs, dsts, tab, asrc, adst, out,
               idxs_v, idxd_v, w_v, rows_v, as_v, ad_v, acc,
               semg0, semg1, semsc, semi):
    c = lax.axis_index("c")
    s = lax.axis_index("s")
    c_off = c * R
    rows_per_tile = R // NSUB
    nv = TW // 16
    semg = (semg0, semg1)

    def zbody(i, carry):
        for r in range(nv):
            rows_v[0, i, pl.ds(r * 16, 16)] = jnp.zeros((16,), jnp.float32)
        return carry
    lax.fori_loop(0, CHUNK, zbody, 0)
    for j in range(rows_per_tile // CHUNK):
        pltpu.sync_copy(
            rows_v.at[0], acc.at[pl.ds(s * rows_per_tile + j * CHUNK, CHUNK)])

    pltpu.sync_copy(asrc, as_v)
    pltpu.sync_copy(adst, ad_v)
    plsc.subcore_barrier()

    def start_idx(k, p):
        base = s * PER_TILE + k * CHUNK
        pltpu.async_copy(srcs.at[pl.ds(base, CHUNK)], idxs_v.at[p], semi)
        pltpu.async_copy(dsts.at[pl.ds(base, CHUNK)], idxd_v.at[p], semi)

    def wait_idx(p):
        pltpu.make_async_copy(srcs.at[pl.ds(0, CHUNK)], idxs_v.at[p],
                              semi).wait()
        pltpu.make_async_copy(dsts.at[pl.ds(0, CHUNK)], idxd_v.at[p],
                              semi).wait()

    def alpha(p):
        for i in range(CHUNK // 16):
            s16 = idxs_v[p, pl.ds(i * 16, 16)] + c_off
            idxs_v[p, pl.ds(i * 16, 16)] = s16
            d16 = idxd_v[p, pl.ds(i * 16, 16)] + c_off
            a = plsc.load_gather(as_v, [s16]) + plsc.load_gather(ad_v, [d16])
            a = jnp.maximum(a, 0.2 * a)
            w_v[p, pl.ds(i * 16, 16)] = jnp.exp(a)

    def start_gather(p):
        pltpu.async_copy(tab.at[idxs_v.at[p]], rows_v.at[p], semg[p])

    def wait_gather(p):
        pltpu.make_async_copy(tab.at[idxs_v.at[p]], rows_v.at[p],
                              semg[p]).wait()

    def start_scatter(p):
        pltpu.async_copy(rows_v.at[p], acc.at[idxd_v.at[p]], semsc, add=True)

    def wait_scatter(p):
        pltpu.make_async_copy(rows_v.at[p], acc.at[idxd_v.at[p]],
                              semsc).wait()

    def mul(p):
        def mul_body(e, carry2):
            wb = plsc.load_gather(w_v.at[p], [jnp.broadcast_to(e, (16,))])
            for r in range(nv):
                rows_v[p, e, pl.ds(r * 16, 16)] = (
                    rows_v[p, e, pl.ds(r * 16, 16)] * wb)
            return carry2
        lax.fori_loop(0, CHUNK, mul_body, 0)

    start_idx(0, 0)
    wait_idx(0)
    alpha(0)
    start_gather(0)

    def pair_body2(j, carry):
        first0 = j == 0

        def half(i, p, q, skip_wait, do_next):
            @pl.when(jnp.logical_not(skip_wait))
            def _():
                wait_scatter(q)

            @pl.when(do_next)
            def _():
                start_idx(i + 1, q)
            wait_gather(p)
            mul(p)
            start_scatter(p)

            @pl.when(do_next)
            def _():
                wait_idx(q)
                alpha(q)
                start_gather(q)

        i0 = 2 * j
        half(i0, 0, 1, first0, jnp.bool_(True))
        half(i0 + 1, 1, 0, jnp.bool_(False),
             j < (NCHUNK // 2 - 1))
        return carry
    lax.fori_loop(0, NCHUNK // 2, pair_body2, 0)
    wait_scatter((NCHUNK - 1) % 2)

    plsc.subcore_barrier()
    pltpu.sync_copy(
        acc.at[pl.ds(s * rows_per_tile, rows_per_tile)],
        out.at[pl.ds(c_off + s * rows_per_tile, rows_per_tile)])


def _edge_pass():
    mesh = plsc.VectorSubcoreMesh(
        core_axis_name="c", subcore_axis_name="s", num_cores=H,
        num_subcores=NSUB)
    return pl.kernel(
        _edge_body,
        out_type=jax.ShapeDtypeStruct((H * R, TW), jnp.float32),
        mesh=mesh,
        scratch_types=[
            pltpu.VMEM((2, CHUNK), jnp.int32),
            pltpu.VMEM((2, CHUNK), jnp.int32),
            pltpu.VMEM((2, CHUNK), jnp.float32),
            pltpu.VMEM((2, CHUNK, TW), jnp.float32),
            pltpu.VMEM((H * R,), jnp.float32),
            pltpu.VMEM((H * R,), jnp.float32),
            pltpu.VMEM_SHARED((R, TW), jnp.float32),
            pltpu.SemaphoreType.DMA,
            pltpu.SemaphoreType.DMA,
            pltpu.SemaphoreType.DMA,
            pltpu.SemaphoreType.DMA,
        ],
        compiler_params=pltpu.CompilerParams(
            needs_layout_passes=False, use_tc_tiling_on_sc=False),
    )


def kernel(x, edge_index, W1, a_src1, a_dst1, b1, gamma, beta, W2, a_src2, a_dst2, b2):
    f32 = jnp.float32
    i32 = jnp.int32
    xp = jnp.zeros((R, D_IN), f32).at[:N].set(x)
    loop = jnp.arange(N, dtype=i32)
    padi = jnp.full((EPAD - E1,), N, dtype=i32)
    srcs = jnp.concatenate([edge_index[0].astype(i32), loop, padi])
    dsts = jnp.concatenate([edge_index[1].astype(i32), loop, padi])

    A1 = jnp.zeros((H * HID, 8), f32)
    A1 = A1.at[:HID, 0].set(a_src1[0]).at[HID:, 1].set(a_src1[1])
    A1 = A1.at[:HID, 2].set(a_dst1[0]).at[HID:, 3].set(a_dst1[1])
    A2 = jnp.zeros((H * OUT, 8), f32)
    A2 = A2.at[:OUT, 0].set(a_src2[0]).at[OUT:, 1].set(a_src2[1])
    A2 = A2.at[:OUT, 2].set(a_dst2[0]).at[OUT:, 3].set(a_dst2[1])

    edge = _edge_pass()

    tab1, al1 = _stage_a(xp, W1, A1)
    asrc1 = jnp.concatenate([al1[:, 0], al1[:, 1]])
    adst1 = jnp.concatenate([al1[:, 2], al1[:, 3]])
    acc1 = edge(srcs, dsts, tab1.reshape(H * R, TW), asrc1, adst1)

    tab2a, tab2b, al2 = _stage_c(acc1.reshape(H, R, TW), b1.reshape(1, -1),
                                 gamma.reshape(1, -1), beta.reshape(1, -1),
                                 W2, A2)
    asrc2 = jnp.concatenate([al2[:, 0], al2[:, 1]])
    adst2 = jnp.concatenate([al2[:, 2], al2[:, 3]])
    acc2a = edge(srcs, dsts, tab2a.reshape(H * R, TW), asrc2, adst2)
    acc2b = edge(srcs, dsts, tab2b.reshape(H * R, TW), asrc2, adst2)

    out = _stage_e(acc2a.reshape(H, R, TW), acc2b.reshape(H, R, TW),
                   b2.reshape(1, -1))
    return out[:N]


# speedup vs baseline: 50.2351x; 1.3098x over previous
import functools

import jax
import jax.numpy as jnp
import numpy as np
from jax import lax
from jax.experimental import pallas as pl
from jax.experimental.pallas import tpu as pltpu
from jax.experimental.pallas import tpu_sc as plsc

N = 10000
E = 320000
D_IN = 128
HID = 64
OUT = 128
H = 2

R = 10240
BLK = 1024
GRID = R // BLK
CHUNK = 128
NSUB = 16
E1 = E + N
EPAD = ((E1 + NSUB * CHUNK - 1) // (NSUB * CHUNK)) * (NSUB * CHUNK)
PER_TILE = EPAD // NSUB
NCHUNK = PER_TILE // CHUNK
TW = 80


def _ones_pad_cols(nrows, width):
    col = lax.broadcasted_iota(jnp.int32, (nrows, width), 1)
    return jnp.where(col == 0, 1.0, 0.0).astype(jnp.float32)


def _mk_tab(h64):
    return jnp.concatenate([h64, _ones_pad_cols(BLK, TW - HID)], axis=1)


def _stage_a_body(x_ref, w_ref, a_ref, tab_ref, al_ref):
    h = jnp.dot(x_ref[...], w_ref[...], preferred_element_type=jnp.float32)
    al_ref[...] = jnp.dot(h, a_ref[...], preferred_element_type=jnp.float32)
    tab_ref[...] = jnp.stack([_mk_tab(h[:, :HID]), _mk_tab(h[:, HID:])])


def _stage_a(xp, W1, A1):
    return pl.pallas_call(
        _stage_a_body,
        grid=(GRID,),
        in_specs=[
            pl.BlockSpec((BLK, D_IN), lambda i: (i, 0)),
            pl.BlockSpec((D_IN, H * HID), lambda i: (0, 0)),
            pl.BlockSpec((H * HID, 8), lambda i: (0, 0)),
        ],
        out_specs=[
            pl.BlockSpec((H, BLK, TW), lambda i: (0, i, 0)),
            pl.BlockSpec((BLK, 8), lambda i: (i, 0)),
        ],
        out_shape=[
            jax.ShapeDtypeStruct((H, R, TW), jnp.float32),
            jax.ShapeDtypeStruct((R, 8), jnp.float32),
        ],
    )(xp, W1, A1)


def _stage_c_body(acc_ref, b1_ref, g_ref, be_ref, w2_ref, a2_ref,
                  taba_ref, tabb_ref, al_ref):
    a0 = acc_ref[0]
    a1 = acc_ref[1]
    h0 = a0[:, :HID] / (a0[:, HID:HID + 1] + 1e-16)
    h1 = a1[:, :HID] / (a1[:, HID:HID + 1] + 1e-16)
    h = jnp.concatenate([h0, h1], axis=1) + b1_ref[...]
    h = h * (g_ref[...] * np.float32(1.0 / np.sqrt(1.0 + 1e-5))) + be_ref[...]
    h = jnp.where(h > 0, h, jnp.exp(h) - 1.0)
    h2 = jnp.dot(h, w2_ref[...], preferred_element_type=jnp.float32)
    al_ref[...] = jnp.dot(h2, a2_ref[...], preferred_element_type=jnp.float32)
    taba_ref[...] = jnp.stack([_mk_tab(h2[:, :HID]),
                               _mk_tab(h2[:, OUT:OUT + HID])])
    tabb_ref[...] = jnp.stack([_mk_tab(h2[:, HID:OUT]),
                               _mk_tab(h2[:, OUT + HID:])])


def _stage_c(acc1, b1, gamma, beta, W2, A2):
    return pl.pallas_call(
        _stage_c_body,
        grid=(GRID,),
        in_specs=[
            pl.BlockSpec((H, BLK, TW), lambda i: (0, i, 0)),
            pl.BlockSpec((1, H * HID), lambda i: (0, 0)),
            pl.BlockSpec((1, H * HID), lambda i: (0, 0)),
            pl.BlockSpec((1, H * HID), lambda i: (0, 0)),
            pl.BlockSpec((H * HID, H * OUT), lambda i: (0, 0)),
            pl.BlockSpec((H * OUT, 8), lambda i: (0, 0)),
        ],
        out_specs=[
            pl.BlockSpec((H, BLK, TW), lambda i: (0, i, 0)),
            pl.BlockSpec((H, BLK, TW), lambda i: (0, i, 0)),
            pl.BlockSpec((BLK, 8), lambda i: (i, 0)),
        ],
        out_shape=[
            jax.ShapeDtypeStruct((H, R, TW), jnp.float32),
            jax.ShapeDtypeStruct((H, R, TW), jnp.float32),
            jax.ShapeDtypeStruct((R, 8), jnp.float32),
        ],
    )(acc1, b1, gamma, beta, W2, A2)


def _stage_e_body(acca_ref, accb_ref, b2_ref, out_ref):
    oa = [acca_ref[c][:, :HID] / (acca_ref[c][:, HID:HID + 1] + 1e-16)
          for c in range(H)]
    ob = [accb_ref[c][:, :HID] / (accb_ref[c][:, HID:HID + 1] + 1e-16)
          for c in range(H)]
    lo = 0.5 * (oa[0] + oa[1])
    hi = 0.5 * (ob[0] + ob[1])
    out_ref[...] = jnp.concatenate([lo, hi], axis=1) + b2_ref[...]


def _stage_e(acc2a, acc2b, b2):
    return pl.pallas_call(
        _stage_e_body,
        grid=(GRID,),
        in_specs=[
            pl.BlockSpec((H, BLK, TW), lambda i: (0, i, 0)),
            pl.BlockSpec((H, BLK, TW), lambda i: (0, i, 0)),
            pl.BlockSpec((1, OUT), lambda i: (0, 0)),
        ],
        out_specs=pl.BlockSpec((BLK, OUT), lambda i: (i, 0)),
        out_shape=jax.ShapeDtypeStruct((R, OUT), jnp.float32),
    )(acc2a, acc2b, b2)


def _edge_body(srcs, dsts, tab, asrc, adst, out,
               idxs_v, idxd_v, w_v, rows_v, as_v, ad_v, acc,
               semg0, semg1, semsc0, semsc1, semi0, semi1, semi2):
    c = lax.axis_index("c")
    s = lax.axis_index("s")
    c_off = c * R
    rows_per_tile = R // NSUB
    nv = TW // 16
    semg = (semg0, semg1)
    semsc = (semsc0, semsc1)
    semi = (semi0, semi1, semi2)

    def zbody(i, carry):
        for r in range(nv):
            rows_v[0, i, pl.ds(r * 16, 16)] = jnp.zeros((16,), jnp.float32)
        return carry
    lax.fori_loop(0, CHUNK, zbody, 0)
    for j in range(rows_per_tile // CHUNK):
        pltpu.sync_copy(
            rows_v.at[0], acc.at[pl.ds(s * rows_per_tile + j * CHUNK, CHUNK)])

    pltpu.sync_copy(asrc, as_v)
    pltpu.sync_copy(adst, ad_v)
    plsc.subcore_barrier()

    def start_idx(k, sl):
        base = s * PER_TILE + k * CHUNK
        pltpu.async_copy(srcs.at[pl.ds(base, CHUNK)], idxs_v.at[sl], semi[sl])
        pltpu.async_copy(dsts.at[pl.ds(base, CHUNK)], idxd_v.at[sl], semi[sl])

    def wait_idx(sl):
        pltpu.make_async_copy(srcs.at[pl.ds(0, CHUNK)], idxs_v.at[sl],
                              semi[sl]).wait()
        pltpu.make_async_copy(dsts.at[pl.ds(0, CHUNK)], idxd_v.at[sl],
                              semi[sl]).wait()

    def alpha(sl, p):
        for i in range(CHUNK // 16):
            s16 = idxs_v[sl, pl.ds(i * 16, 16)] + c_off
            idxs_v[sl, pl.ds(i * 16, 16)] = s16
            d16 = idxd_v[sl, pl.ds(i * 16, 16)] + c_off
            a = plsc.load_gather(as_v, [s16]) + plsc.load_gather(ad_v, [d16])
            a = jnp.maximum(a, 0.2 * a)
            w_v[p, pl.ds(i * 16, 16)] = jnp.exp(a)

    def start_gather(sl, p):
        pltpu.async_copy(tab.at[idxs_v.at[sl]], rows_v.at[p], semg[p])

    def wait_gather(sl, p):
        pltpu.make_async_copy(tab.at[idxs_v.at[sl]], rows_v.at[p],
                              semg[p]).wait()

    def start_scatter(sl, p):
        pltpu.async_copy(rows_v.at[p], acc.at[idxd_v.at[sl]], semsc[p],
                         add=True)

    def wait_scatter(sl, p):
        pltpu.make_async_copy(rows_v.at[p], acc.at[idxd_v.at[sl]],
                              semsc[p]).wait()

    def mul(p):
        def mul_body(e, carry2):
            wb = plsc.load_gather(w_v.at[p], [jnp.broadcast_to(e, (16,))])
            for r in range(nv):
                rows_v[p, e, pl.ds(r * 16, 16)] = (
                    rows_v[p, e, pl.ds(r * 16, 16)] * wb)
            return carry2
        lax.fori_loop(0, CHUNK, mul_body, 0)

    start_idx(0, 0)
    start_idx(1, 1)
    wait_idx(0)
    alpha(0, 0)
    start_gather(0, 0)

    def six_body(j, carry):
        for t in range(6):
            i = 6 * j + t
            p = t % 2
            q = 1 - p
            sl = t % 3
            sl1 = (t + 1) % 3
            sl2 = (t + 2) % 3
            if t == 0:
                @pl.when(j > 0)
                def _():
                    wait_scatter(sl2, q)
            else:
                wait_scatter(sl2, q)
            if t >= 4:
                @pl.when(j < NCHUNK // 6 - 1)
                def _():
                    start_idx(i + 2, sl2)
            else:
                start_idx(i + 2, sl2)
            if t == 5:
                @pl.when(j < NCHUNK // 6 - 1)
                def _():
                    wait_idx(sl1)
                    alpha(sl1, q)
                    start_gather(sl1, q)
            else:
                wait_idx(sl1)
                alpha(sl1, q)
                start_gather(sl1, q)
            wait_gather(sl, p)
            mul(p)
            start_scatter(sl, p)
        return carry
    lax.fori_loop(0, NCHUNK // 6, six_body, 0)
    wait_scatter((NCHUNK - 1) % 3, (NCHUNK - 1) % 2)

    plsc.subcore_barrier()
    pltpu.sync_copy(
        acc.at[pl.ds(s * rows_per_tile, rows_per_tile)],
        out.at[pl.ds(c_off + s * rows_per_tile, rows_per_tile)])


def _edge_pass():
    mesh = plsc.VectorSubcoreMesh(
        core_axis_name="c", subcore_axis_name="s", num_cores=H,
        num_subcores=NSUB)
    return pl.kernel(
        _edge_body,
        out_type=jax.ShapeDtypeStruct((H * R, TW), jnp.float32),
        mesh=mesh,
        scratch_types=[
            pltpu.VMEM((3, CHUNK), jnp.int32),
            pltpu.VMEM((3, CHUNK), jnp.int32),
            pltpu.VMEM((2, CHUNK), jnp.float32),
            pltpu.VMEM((2, CHUNK, TW), jnp.float32),
            pltpu.VMEM((H * R,), jnp.float32),
            pltpu.VMEM((H * R,), jnp.float32),
            pltpu.VMEM_SHARED((R, TW), jnp.float32),
            pltpu.SemaphoreType.DMA,
            pltpu.SemaphoreType.DMA,
            pltpu.SemaphoreType.DMA,
            pltpu.SemaphoreType.DMA,
            pltpu.SemaphoreType.DMA,
            pltpu.SemaphoreType.DMA,
            pltpu.SemaphoreType.DMA,
        ],
        compiler_params=pltpu.CompilerParams(
            needs_layout_passes=False, use_tc_tiling_on_sc=False),
    )


def kernel(x, edge_index, W1, a_src1, a_dst1, b1, gamma, beta, W2, a_src2, a_dst2, b2):
    f32 = jnp.float32
    i32 = jnp.int32
    xp = jnp.zeros((R, D_IN), f32).at[:N].set(x)
    loop = jnp.arange(N, dtype=i32)
    padi = jnp.full((EPAD - E1,), N, dtype=i32)
    srcs = jnp.concatenate([edge_index[0].astype(i32), loop, padi])
    dsts = jnp.concatenate([edge_index[1].astype(i32), loop, padi])

    A1 = jnp.zeros((H * HID, 8), f32)
    A1 = A1.at[:HID, 0].set(a_src1[0]).at[HID:, 1].set(a_src1[1])
    A1 = A1.at[:HID, 2].set(a_dst1[0]).at[HID:, 3].set(a_dst1[1])
    A2 = jnp.zeros((H * OUT, 8), f32)
    A2 = A2.at[:OUT, 0].set(a_src2[0]).at[OUT:, 1].set(a_src2[1])
    A2 = A2.at[:OUT, 2].set(a_dst2[0]).at[OUT:, 3].set(a_dst2[1])

    edge = _edge_pass()

    tab1, al1 = _stage_a(xp, W1, A1)
    asrc1 = jnp.concatenate([al1[:, 0], al1[:, 1]])
    adst1 = jnp.concatenate([al1[:, 2], al1[:, 3]])
    acc1 = edge(srcs, dsts, tab1.reshape(H * R, TW), asrc1, adst1)

    tab2a, tab2b, al2 = _stage_c(acc1.reshape(H, R, TW), b1.reshape(1, -1),
                                 gamma.reshape(1, -1), beta.reshape(1, -1),
                                 W2, A2)
    asrc2 = jnp.concatenate([al2[:, 0], al2[:, 1]])
    adst2 = jnp.concatenate([al2[:, 2], al2[:, 3]])
    acc2a = edge(srcs, dsts, tab2a.reshape(H * R, TW), asrc2, adst2)
    acc2b = edge(srcs, dsts, tab2b.reshape(H * R, TW), asrc2, adst2)

    out = _stage_e(acc2a.reshape(H, R, TW), acc2b.reshape(H, R, TW),
                   b2.reshape(1, -1))
    return out[:N]
